# baseline (speedup 1.0000x reference)
import functools

import jax
import jax.numpy as jnp
from jax import lax
from jax.experimental import pallas as pl
from jax.experimental.pallas import tpu as pltpu
from jax.experimental.pallas import tpu_sc as plsc

_THIRD = 1.0 / 3.0
_NC = 2
_NW = 32
_R = 240
_NGRP = _R // 16
_CPB = 8400 // _R
_BPW = 2
_NCHUNK = _CPB * _BPW

_TRIPLES = [(s, s + 10, s + 20) for s in list(range(10)) + list(range(30, 40))]


def _sc_stats_kernel(x_hbm, out_hbm, buf0, buf1, acc, sem0, sem1):
    cid = lax.axis_index("c")
    sid = lax.axis_index("s")
    wid = sid * _NC + cid
    b0 = wid * _BPW

    zeros16 = jnp.zeros((16,), jnp.float32)
    ninf16 = jnp.full((16,), -jnp.inf, jnp.float32)
    liota = lax.iota(jnp.int32, 16)

    for j in range(20):
        for r in range(4):
            acc[r, pl.ds(j * 16, 16)] = zeros16
        for r in range(4, 6):
            acc[r, pl.ds(j * 16, 16)] = ninf16

    def chunk_src(k):
        batch = b0 + k // _CPB
        row0 = (k % _CPB) * _R
        return x_hbm.at[batch, pl.ds(row0, _R), :]

    idx_cols = [(jnp.full((16,), s, jnp.int32),
                 jnp.full((16,), t1, jnp.int32),
                 jnp.full((16,), t2, jnp.int32)) for s, t1, t2 in _TRIPLES]

    def process(buf):
        def grp(g, _):
            ridx = g * 16 + liota
            for j, (cs, c1, c2) in enumerate(idx_cols):
                xs = plsc.load_gather(buf, [ridx, cs])
                x1 = plsc.load_gather(buf, [ridx, c1])
                x2 = plsc.load_gather(buf, [ridx, c2])
                ss = 1.0 / (1.0 + jnp.exp(-xs))
                s1 = 1.0 / (1.0 + jnp.exp(-x1))
                s2 = 1.0 / (1.0 + jnp.exp(-x2))
                om = 1.0 - ss
                v1 = s1 * om
                v2 = s2 * om
                ve = s1 * s2
                o = j * 16
                plsc.addupdate(acc.at[0, pl.ds(o, 16)], ss * ss * ss)
                plsc.addupdate(acc.at[1, pl.ds(o, 16)], v1 * v1 * v1)
                plsc.addupdate(acc.at[2, pl.ds(o, 16)], v2 * v2 * v2)
                plsc.addupdate(acc.at[3, pl.ds(o, 16)], ve * ve * ve)
                acc[4, pl.ds(o, 16)] = jnp.maximum(acc[4, pl.ds(o, 16)], x1)
                acc[5, pl.ds(o, 16)] = jnp.maximum(acc[5, pl.ds(o, 16)], x2)
            return 0

        lax.fori_loop(0, _NGRP, grp, 0)

    pltpu.async_copy(chunk_src(0), buf0, sem0).wait()

    def two_chunks(i, _):
        k0 = i * 2
        cp1 = pltpu.async_copy(chunk_src(k0 + 1), buf1, sem1)
        process(buf0)
        cp1.wait()
        nxt = jnp.minimum(k0 + 2, _NCHUNK - 1)
        cp0 = pltpu.async_copy(chunk_src(nxt), buf0, sem0)
        process(buf1)
        cp0.wait()
        return 0

    lax.fori_loop(0, _NCHUNK // 2, two_chunks, 0)

    pltpu.sync_copy(acc, out_hbm.at[wid])


def _combine_kernel(y_ref, out_ref, *, n_rows):
    y = y_ref[...]
    sums = jnp.sum(y[:, 0:4, :], axis=0)
    maxs = jnp.max(y[:, 4:6, :], axis=0)
    lane = lax.broadcasted_iota(jnp.int32, (320, 32), 0) // 16
    col = lax.broadcasted_iota(jnp.int32, (320, 32), 1)
    sel = (lane == col).astype(jnp.float32)
    sums20 = jax.lax.dot_general(
        sums, sel, (((1,), (0,)), ((), ())),
        preferred_element_type=jnp.float32,
        precision=jax.lax.Precision.HIGHEST)
    glane = lax.broadcasted_iota(jnp.int32, (2, 320), 1) // 16
    mcols = [jnp.max(jnp.where(glane == j, maxs, -jnp.inf),
                     axis=1, keepdims=True) for j in range(20)]
    maxs20 = jnp.concatenate(mcols, axis=1)
    inv_n = 1.0 / n_rows
    roots = (sums20[:, 0:20] * inv_n) ** _THIRD
    p3r = roots[0:1, :]
    qr = roots[1:2, :] + roots[2:3, :]
    er = roots[3:4, :]
    m1 = 1.0 / (1.0 + jnp.exp(-maxs20[0:1, :]))
    m2 = 1.0 / (1.0 + jnp.exp(-maxs20[1:2, :]))
    m = jnp.maximum(m1, m2)
    picked = 0.1 * ((1.0 - m) * p3r + er) + 0.05 * qr
    out_ref[...] = jnp.sum(picked, axis=1, keepdims=True)[0:1, 0:1]


def kernel(pred_scores, target_scores):
    del target_scores
    b, a, c = pred_scores.shape
    n_rows = b * a

    mesh = plsc.VectorSubcoreMesh(core_axis_name="c", subcore_axis_name="s")
    stats = pl.kernel(
        _sc_stats_kernel,
        mesh=mesh,
        compiler_params=pltpu.CompilerParams(needs_layout_passes=False),
        out_type=jax.ShapeDtypeStruct((_NW, 8, 320), jnp.float32),
        scratch_types=[
            pltpu.VMEM((_R, c), jnp.float32),
            pltpu.VMEM((_R, c), jnp.float32),
            pltpu.VMEM((8, 320), jnp.float32),
            pltpu.SemaphoreType.DMA,
            pltpu.SemaphoreType.DMA,
        ],
    )(pred_scores)

    out = pl.pallas_call(
        functools.partial(_combine_kernel, n_rows=n_rows),
        out_shape=jax.ShapeDtypeStruct((1, 1), jnp.float32),
    )(stats)
    return out.reshape(())


# speedup vs baseline: 3.0606x; 3.0606x over previous
import functools

import jax
import jax.numpy as jnp
from jax import lax
from jax.experimental import pallas as pl
from jax.experimental.pallas import tpu as pltpu
from jax.experimental.pallas import tpu_sc as plsc

_THIRD = 1.0 / 3.0

_NC = 2
_NW = 32
_SC_BATCHES = 32
_R = 240
_NGRP = _R // 16
_NCHUNK = 8400 // _R

_TC_BB = 4

_TRIPLES = [(s, s + 10, s + 20) for s in list(range(10)) + list(range(30, 40))]


def _sc_stats_kernel(x_hbm, out_hbm, buf0, buf1, acc, sem0, sem1):
    cid = lax.axis_index("c")
    sid = lax.axis_index("s")
    wid = sid * _NC + cid
    batch = wid

    zeros16 = jnp.zeros((16,), jnp.float32)
    ninf16 = jnp.full((16,), -jnp.inf, jnp.float32)
    liota = lax.iota(jnp.int32, 16)

    for r in range(4):
        acc[r, pl.ds(0, 16)] = zeros16
        acc[r, pl.ds(16, 16)] = zeros16
    for r in range(4, 6):
        acc[r, pl.ds(0, 16)] = ninf16
        acc[r, pl.ds(16, 16)] = ninf16

    def chunk_src(k):
        return x_hbm.at[batch, pl.ds(k * _R, _R), :]

    def process(buf):
        for j, (s, t1, t2) in enumerate(_TRIPLES):
            cs = jnp.full((16,), s, jnp.int32)
            c1 = jnp.full((16,), t1, jnp.int32)
            c2 = jnp.full((16,), t2, jnp.int32)

            def grp(g, carry):
                p, q1, q2, e, m1, m2 = carry
                ridx = g * 16 + liota
                xs = plsc.load_gather(buf, [ridx, cs])
                x1 = plsc.load_gather(buf, [ridx, c1])
                x2 = plsc.load_gather(buf, [ridx, c2])
                ss = 1.0 / (1.0 + jnp.exp(-xs))
                s1 = 1.0 / (1.0 + jnp.exp(-x1))
                s2 = 1.0 / (1.0 + jnp.exp(-x2))
                om = 1.0 - ss
                v1 = s1 * om
                v2 = s2 * om
                ve = s1 * s2
                p = p + ss * ss * ss
                q1 = q1 + v1 * v1 * v1
                q2 = q2 + v2 * v2 * v2
                e = e + ve * ve * ve
                m1 = jnp.maximum(m1, x1)
                m2 = jnp.maximum(m2, x2)
                return p, q1, q2, e, m1, m2

            p, q1, q2, e, m1, m2 = lax.fori_loop(
                0, _NGRP, grp,
                (zeros16, zeros16, zeros16, zeros16, ninf16, ninf16))

            half = 16 * (j // 16)
            jj = j % 16
            for r, v in ((0, jnp.sum(p)), (1, jnp.sum(q1)),
                         (2, jnp.sum(q2)), (3, jnp.sum(e))):
                cur = acc[r, pl.ds(half, 16)]
                acc[r, pl.ds(half, 16)] = cur + jnp.where(liota == jj, v, 0.0)
            for r, v in ((4, jnp.max(m1)), (5, jnp.max(m2))):
                cur = acc[r, pl.ds(half, 16)]
                acc[r, pl.ds(half, 16)] = jnp.maximum(
                    cur, jnp.where(liota == jj, v, -jnp.inf))

    pltpu.async_copy(chunk_src(0), buf0, sem0).wait()

    def two_chunks(i, _):
        k0 = i * 2
        cp1 = pltpu.async_copy(chunk_src(k0 + 1), buf1, sem1)
        process(buf0)
        cp1.wait()
        nxt = jnp.minimum(k0 + 2, _NCHUNK - 1)
        cp0 = pltpu.async_copy(chunk_src(nxt), buf0, sem0)
        process(buf1)
        cp0.wait()
        return 0

    lax.fori_loop(0, _NCHUNK // 2, two_chunks, 0)
    if _NCHUNK % 2:
        process(buf0)

    pltpu.sync_copy(acc, out_hbm.at[wid])


def _tc_stats_kernel(x_ref, acc_out, g_out, h_out, *, n_steps):
    pi = pl.program_id(0)

    @pl.when(pi == 0)
    def _init():
        acc_out[0:1, :] = jnp.zeros((1, acc_out.shape[1]), jnp.float32)
        acc_out[1:2, :] = jnp.full((1, acc_out.shape[1]), -jnp.inf,
                                   jnp.float32)
        acc_out[2:8, :] = jnp.zeros((6, acc_out.shape[1]), jnp.float32)
        g_out[...] = jnp.zeros_like(g_out)
        h_out[...] = jnp.zeros_like(h_out)

    dn = (((0,), (0,)), ((), ()))
    for i in range(_TC_BB):
        x = x_ref[i]
        a = jnp.exp(-x)
        sig = 1.0 / (1.0 + a)
        om = a * sig
        s2 = sig * sig
        a3 = s2 * sig
        o2 = om * om
        b3 = o2 * om
        a3b = a3.astype(jnp.bfloat16)
        b3b = b3.astype(jnp.bfloat16)
        g_out[...] += jax.lax.dot_general(
            a3b, b3b, dn, preferred_element_type=jnp.float32)
        h_out[...] += jax.lax.dot_general(
            a3b, a3b, dn, preferred_element_type=jnp.float32)
        acc_out[0:1, :] += jnp.sum(a3, axis=0, keepdims=True)
        acc_out[1:2, :] = jnp.maximum(acc_out[1:2, :],
                                      jnp.max(x, axis=0, keepdims=True))


def _combine_kernel(acc_ref, g_ref, h_ref, y_ref, out_ref, *, n_rows):
    nc = acc_ref.shape[1]
    rows = lax.broadcasted_iota(jnp.int32, (nc, nc), 0)
    cols = lax.broadcasted_iota(jnp.int32, (nc, nc), 1)
    q1v = jnp.sum(jnp.where(rows == cols + 10, g_ref[...], 0.0),
                  axis=0, keepdims=True)
    q2v = jnp.sum(jnp.where(rows == cols + 20, g_ref[...], 0.0),
                  axis=0, keepdims=True)
    evc = jnp.sum(jnp.where(rows + 10 == cols, h_ref[...], 0.0),
                  axis=0, keepdims=True)
    ev = jnp.concatenate([evc[:, 20:], evc[:, :20]], axis=1)
    pv = acc_ref[0:1, :]
    mx = acc_ref[1:2, :]
    m1v = jnp.concatenate([mx[:, 10:], mx[:, :10]], axis=1)
    m2v = jnp.concatenate([mx[:, 20:], mx[:, :20]], axis=1)

    y = y_ref[...]
    ssum = jnp.sum(y[:, 0:4, :], axis=0)
    smax = jnp.max(y[:, 4:6, :], axis=0)
    lane320 = lax.broadcasted_iota(jnp.int32, (4, 320), 1) // 16
    lane80 = lax.broadcasted_iota(jnp.int32, (1, nc), 1)

    def per_class(vals, is_max):
        outs = []
        for j in range(20):
            grp_lane = lane320[0:1, :] == j
            if is_max:
                outs.append(jnp.max(jnp.where(grp_lane, vals, -jnp.inf),
                                    axis=1, keepdims=True))
            else:
                outs.append(jnp.sum(jnp.where(grp_lane, vals, 0.0),
                                    axis=1, keepdims=True))
        v20 = jnp.concatenate(outs, axis=1)
        fill = jnp.full((1, 20), -jnp.inf if is_max else 0.0, jnp.float32)
        return jnp.concatenate(
            [v20[:, 0:10], fill, v20[:, 10:20], fill, fill], axis=1)

    sc_p = per_class(ssum[0:1, :], False)
    sc_q1 = per_class(ssum[1:2, :], False)
    sc_q2 = per_class(ssum[2:3, :], False)
    sc_e = per_class(ssum[3:4, :], False)
    sc_m1 = per_class(smax[0:1, :], True)
    sc_m2 = per_class(smax[1:2, :], True)

    inv_n = 1.0 / n_rows
    p3r = ((pv + sc_p) * inv_n) ** _THIRD
    q1r = ((q1v + sc_q1) * inv_n) ** _THIRD
    q2r = ((q2v + sc_q2) * inv_n) ** _THIRD
    er = ((ev + sc_e) * inv_n) ** _THIRD
    m1 = 1.0 / (1.0 + jnp.exp(-jnp.maximum(m1v, sc_m1)))
    m2 = 1.0 / (1.0 + jnp.exp(-jnp.maximum(m2v, sc_m2)))
    m = jnp.maximum(m1, m2)
    is_src = jnp.logical_or(lane80 < 10,
                            jnp.logical_and(lane80 >= 30, lane80 < 40))
    picked = jnp.where(is_src,
                       0.1 * ((1.0 - m) * p3r + er) + 0.05 * (q1r + q2r),
                       0.0)
    out_ref[...] = jnp.sum(picked, axis=1, keepdims=True)[0:1, 0:1]


def kernel(pred_scores, target_scores):
    del target_scores
    b, a, c = pred_scores.shape
    n_rows = b * a

    mesh = plsc.VectorSubcoreMesh(core_axis_name="c", subcore_axis_name="s")
    sc_stats = pl.kernel(
        _sc_stats_kernel,
        mesh=mesh,
        compiler_params=pltpu.CompilerParams(needs_layout_passes=False),
        out_type=jax.ShapeDtypeStruct((_NW, 8, 320), jnp.float32),
        scratch_types=[
            pltpu.VMEM((_R, c), jnp.float32),
            pltpu.VMEM((_R, c), jnp.float32),
            pltpu.VMEM((8, 320), jnp.float32),
            pltpu.SemaphoreType.DMA,
            pltpu.SemaphoreType.DMA,
        ],
    )(pred_scores)

    tc_steps = (b - _SC_BATCHES) // _TC_BB
    off = _SC_BATCHES // _TC_BB
    acc, g, h = pl.pallas_call(
        functools.partial(_tc_stats_kernel, n_steps=tc_steps),
        grid=(tc_steps,),
        in_specs=[pl.BlockSpec((_TC_BB, a, c), lambda i: (i + off, 0, 0))],
        out_specs=[
            pl.BlockSpec((8, c), lambda i: (0, 0)),
            pl.BlockSpec((c, c), lambda i: (0, 0)),
            pl.BlockSpec((c, c), lambda i: (0, 0)),
        ],
        out_shape=[
            jax.ShapeDtypeStruct((8, c), jnp.float32),
            jax.ShapeDtypeStruct((c, c), jnp.float32),
            jax.ShapeDtypeStruct((c, c), jnp.float32),
        ],
    )(pred_scores)

    out = pl.pallas_call(
        functools.partial(_combine_kernel, n_rows=n_rows),
        out_shape=jax.ShapeDtypeStruct((1, 1), jnp.float32),
    )(acc, g, h, sc_stats)
    return out.reshape(())


# speedup vs baseline: 3.0699x; 1.0030x over previous
import functools

import jax
import jax.numpy as jnp
from jax import lax
from jax.experimental import pallas as pl
from jax.experimental.pallas import tpu as pltpu
from jax.experimental.pallas import tpu_sc as plsc

_THIRD = 1.0 / 3.0

_NC = 2
_NW = 32
_SC_BATCHES = 32
_R = 240
_NGRP = _R // 16
_NCHUNK = 8400 // _R

_TC_BB = 4

_TRIPLES = [(s, s + 10, s + 20) for s in list(range(10)) + list(range(30, 40))]


def _sc_stats_kernel(x_hbm, out_hbm, buf0, buf1, acc, sem0, sem1):
    cid = lax.axis_index("c")
    sid = lax.axis_index("s")
    wid = sid * _NC + cid
    batch = wid

    zeros16 = jnp.zeros((16,), jnp.float32)
    ninf16 = jnp.full((16,), -jnp.inf, jnp.float32)
    liota = lax.iota(jnp.int32, 16)

    for r in range(4):
        acc[r, pl.ds(0, 16)] = zeros16
        acc[r, pl.ds(16, 16)] = zeros16
    for r in range(4, 6):
        acc[r, pl.ds(0, 16)] = ninf16
        acc[r, pl.ds(16, 16)] = ninf16

    def chunk_src(k):
        return x_hbm.at[batch, pl.ds(k * _R, _R), :]

    def process(buf):
        for j, (s, t1, t2) in enumerate(_TRIPLES):
            cs = jnp.full((16,), s, jnp.int32)
            c1 = jnp.full((16,), t1, jnp.int32)
            c2 = jnp.full((16,), t2, jnp.int32)

            def grp(g, carry):
                p, q1, q2, e, m1, m2 = carry
                ridx = g * 16 + liota
                xs = plsc.load_gather(buf, [ridx, cs])
                x1 = plsc.load_gather(buf, [ridx, c1])
                x2 = plsc.load_gather(buf, [ridx, c2])
                ss = 1.0 / (1.0 + jnp.exp(-xs))
                s1 = 1.0 / (1.0 + jnp.exp(-x1))
                s2 = 1.0 / (1.0 + jnp.exp(-x2))
                om = 1.0 - ss
                v1 = s1 * om
                v2 = s2 * om
                ve = s1 * s2
                p = p + ss * ss * ss
                q1 = q1 + v1 * v1 * v1
                q2 = q2 + v2 * v2 * v2
                e = e + ve * ve * ve
                m1 = jnp.maximum(m1, x1)
                m2 = jnp.maximum(m2, x2)
                return p, q1, q2, e, m1, m2

            p, q1, q2, e, m1, m2 = lax.fori_loop(
                0, _NGRP, grp,
                (zeros16, zeros16, zeros16, zeros16, ninf16, ninf16))

            half = 16 * (j // 16)
            jj = j % 16
            for r, v in ((0, jnp.sum(p)), (1, jnp.sum(q1)),
                         (2, jnp.sum(q2)), (3, jnp.sum(e))):
                cur = acc[r, pl.ds(half, 16)]
                acc[r, pl.ds(half, 16)] = cur + jnp.where(liota == jj, v, 0.0)
            for r, v in ((4, jnp.max(m1)), (5, jnp.max(m2))):
                cur = acc[r, pl.ds(half, 16)]
                acc[r, pl.ds(half, 16)] = jnp.maximum(
                    cur, jnp.where(liota == jj, v, -jnp.inf))

    pltpu.async_copy(chunk_src(0), buf0, sem0).wait()

    def two_chunks(i, _):
        k0 = i * 2
        cp1 = pltpu.async_copy(chunk_src(k0 + 1), buf1, sem1)
        process(buf0)
        cp1.wait()
        nxt = jnp.minimum(k0 + 2, _NCHUNK - 1)
        cp0 = pltpu.async_copy(chunk_src(nxt), buf0, sem0)
        process(buf1)
        cp0.wait()
        return 0

    lax.fori_loop(0, _NCHUNK // 2, two_chunks, 0)
    if _NCHUNK % 2:
        process(buf0)

    pltpu.sync_copy(acc, out_hbm.at[wid])


def _tc_stats_kernel(x_ref, acc_out, g_out, h_out, *, n_steps):
    pi = pl.program_id(0)

    @pl.when(pi == 0)
    def _init():
        acc_out[0:1, :] = jnp.zeros((1, acc_out.shape[1]), jnp.float32)
        acc_out[1:2, :] = jnp.full((1, acc_out.shape[1]), -jnp.inf,
                                   jnp.float32)
        acc_out[2:8, :] = jnp.zeros((6, acc_out.shape[1]), jnp.float32)
        g_out[...] = jnp.zeros_like(g_out)
        h_out[...] = jnp.zeros_like(h_out)

    dn = (((0,), (0,)), ((), ()))
    for i in range(_TC_BB):
        x = x_ref[i]
        a = jnp.exp(-x)
        sig = 1.0 / (1.0 + a)
        om = a * sig
        s2 = sig * sig
        a3 = s2 * sig
        o2 = om * om
        b3 = o2 * om
        a3b = a3.astype(jnp.bfloat16)
        b3b = b3.astype(jnp.bfloat16)
        g_out[...] += jax.lax.dot_general(
            a3b, b3b, dn, preferred_element_type=jnp.float32)
        h_out[...] += jax.lax.dot_general(
            a3b, a3b, dn, preferred_element_type=jnp.float32)
        acc_out[0:1, :] += jnp.sum(a3, axis=0, keepdims=True)
        acc_out[1:2, :] = jnp.maximum(acc_out[1:2, :],
                                      jnp.max(x, axis=0, keepdims=True))


def _combine_kernel(acc_ref, g_ref, h_ref, y_ref, out_ref, *, n_rows):
    nc = acc_ref.shape[1]
    rows = lax.broadcasted_iota(jnp.int32, (nc, nc), 0)
    cols = lax.broadcasted_iota(jnp.int32, (nc, nc), 1)
    q1v = jnp.sum(jnp.where(rows == cols + 10, g_ref[...], 0.0),
                  axis=0, keepdims=True)
    q2v = jnp.sum(jnp.where(rows == cols + 20, g_ref[...], 0.0),
                  axis=0, keepdims=True)
    evc = jnp.sum(jnp.where(rows + 10 == cols, h_ref[...], 0.0),
                  axis=0, keepdims=True)
    ev = jnp.concatenate([evc[:, 20:], evc[:, :20]], axis=1)
    pv = acc_ref[0:1, :]
    mx = acc_ref[1:2, :]
    m1v = jnp.concatenate([mx[:, 10:], mx[:, :10]], axis=1)
    m2v = jnp.concatenate([mx[:, 20:], mx[:, :20]], axis=1)

    y = y_ref[...]
    ssum = jnp.sum(y[:, 0:4, :], axis=0)
    smax = jnp.max(y[:, 4:6, :], axis=0)
    lane80 = lax.broadcasted_iota(jnp.int32, (1, nc), 1)

    def per_class(vals, is_max):
        fill = jnp.full((1, 20), -jnp.inf if is_max else 0.0, jnp.float32)
        return jnp.concatenate(
            [vals[:, 0:10], fill, vals[:, 10:20], fill, fill], axis=1)

    sc_p = per_class(ssum[0:1, :], False)
    sc_q1 = per_class(ssum[1:2, :], False)
    sc_q2 = per_class(ssum[2:3, :], False)
    sc_e = per_class(ssum[3:4, :], False)
    sc_m1 = per_class(smax[0:1, :], True)
    sc_m2 = per_class(smax[1:2, :], True)

    inv_n = 1.0 / n_rows
    p3r = ((pv + sc_p) * inv_n) ** _THIRD
    q1r = ((q1v + sc_q1) * inv_n) ** _THIRD
    q2r = ((q2v + sc_q2) * inv_n) ** _THIRD
    er = ((ev + sc_e) * inv_n) ** _THIRD
    m1 = 1.0 / (1.0 + jnp.exp(-jnp.maximum(m1v, sc_m1)))
    m2 = 1.0 / (1.0 + jnp.exp(-jnp.maximum(m2v, sc_m2)))
    m = jnp.maximum(m1, m2)
    is_src = jnp.logical_or(lane80 < 10,
                            jnp.logical_and(lane80 >= 30, lane80 < 40))
    picked = jnp.where(is_src,
                       0.1 * ((1.0 - m) * p3r + er) + 0.05 * (q1r + q2r),
                       0.0)
    out_ref[...] = jnp.sum(picked, axis=1, keepdims=True)[0:1, 0:1]


def kernel(pred_scores, target_scores):
    del target_scores
    b, a, c = pred_scores.shape
    n_rows = b * a

    mesh = plsc.VectorSubcoreMesh(core_axis_name="c", subcore_axis_name="s")
    sc_stats = pl.kernel(
        _sc_stats_kernel,
        mesh=mesh,
        compiler_params=pltpu.CompilerParams(needs_layout_passes=False),
        out_type=jax.ShapeDtypeStruct((_NW, 8, 32), jnp.float32),
        scratch_types=[
            pltpu.VMEM((_R, c), jnp.float32),
            pltpu.VMEM((_R, c), jnp.float32),
            pltpu.VMEM((8, 32), jnp.float32),
            pltpu.SemaphoreType.DMA,
            pltpu.SemaphoreType.DMA,
        ],
    )(pred_scores)

    tc_steps = (b - _SC_BATCHES) // _TC_BB
    off = _SC_BATCHES // _TC_BB
    acc, g, h = pl.pallas_call(
        functools.partial(_tc_stats_kernel, n_steps=tc_steps),
        grid=(tc_steps,),
        in_specs=[pl.BlockSpec((_TC_BB, a, c), lambda i: (i + off, 0, 0))],
        out_specs=[
            pl.BlockSpec((8, c), lambda i: (0, 0)),
            pl.BlockSpec((c, c), lambda i: (0, 0)),
            pl.BlockSpec((c, c), lambda i: (0, 0)),
        ],
        out_shape=[
            jax.ShapeDtypeStruct((8, c), jnp.float32),
            jax.ShapeDtypeStruct((c, c), jnp.float32),
            jax.ShapeDtypeStruct((c, c), jnp.float32),
        ],
    )(pred_scores)

    out = pl.pallas_call(
        functools.partial(_combine_kernel, n_rows=n_rows),
        out_shape=jax.ShapeDtypeStruct((1, 1), jnp.float32),
    )(acc, g, h, sc_stats)
    return out.reshape(())


# speedup vs baseline: 4.3135x; 1.4051x over previous
import functools

import jax
import jax.numpy as jnp
from jax import lax
from jax.experimental import pallas as pl
from jax.experimental.pallas import tpu as pltpu
from jax.experimental.pallas import tpu_sc as plsc

_THIRD = 1.0 / 3.0

_NC = 2
_NW = 32
_SC_ROWS = 1200
_R = 240
_NGRP = _R // 16
_CPB = _SC_ROWS // _R
_NCHUNK = 2 * _CPB

_TC_BB = 8
_TC_AB = 1200

_TRIPLES = [(s, s + 10, s + 20) for s in list(range(10)) + list(range(30, 40))]


def _sc_stats_kernel(x_hbm, out_hbm, buf0, buf1, acc, sem0, sem1):
    cid = lax.axis_index("c")
    sid = lax.axis_index("s")
    wid = sid * _NC + cid
    b0 = wid * 2

    zeros16 = jnp.zeros((16,), jnp.float32)
    ninf16 = jnp.full((16,), -jnp.inf, jnp.float32)
    liota = lax.iota(jnp.int32, 16)

    for r in range(4):
        acc[r, pl.ds(0, 16)] = zeros16
        acc[r, pl.ds(16, 16)] = zeros16
    for r in range(4, 6):
        acc[r, pl.ds(0, 16)] = ninf16
        acc[r, pl.ds(16, 16)] = ninf16

    def chunk_src(k):
        batch = b0 + k // _CPB
        row0 = (k % _CPB) * _R
        return x_hbm.at[batch, pl.ds(row0, _R), :]

    def process(buf):
        for j, (s, t1, t2) in enumerate(_TRIPLES):
            cs = jnp.full((16,), s, jnp.int32)
            c1 = jnp.full((16,), t1, jnp.int32)
            c2 = jnp.full((16,), t2, jnp.int32)

            def grp(g, carry):
                p, q1, q2, e, m1, m2 = carry
                ridx = g * 16 + liota
                xs = plsc.load_gather(buf, [ridx, cs])
                x1 = plsc.load_gather(buf, [ridx, c1])
                x2 = plsc.load_gather(buf, [ridx, c2])
                ss = 1.0 / (1.0 + jnp.exp(-xs))
                s1 = 1.0 / (1.0 + jnp.exp(-x1))
                s2 = 1.0 / (1.0 + jnp.exp(-x2))
                om = 1.0 - ss
                v1 = s1 * om
                v2 = s2 * om
                ve = s1 * s2
                p = p + ss * ss * ss
                q1 = q1 + v1 * v1 * v1
                q2 = q2 + v2 * v2 * v2
                e = e + ve * ve * ve
                m1 = jnp.maximum(m1, x1)
                m2 = jnp.maximum(m2, x2)
                return p, q1, q2, e, m1, m2

            p, q1, q2, e, m1, m2 = lax.fori_loop(
                0, _NGRP, grp,
                (zeros16, zeros16, zeros16, zeros16, ninf16, ninf16))

            half = 16 * (j // 16)
            jj = j % 16
            for r, v in ((0, jnp.sum(p)), (1, jnp.sum(q1)),
                         (2, jnp.sum(q2)), (3, jnp.sum(e))):
                cur = acc[r, pl.ds(half, 16)]
                acc[r, pl.ds(half, 16)] = cur + jnp.where(liota == jj, v, 0.0)
            for r, v in ((4, jnp.max(m1)), (5, jnp.max(m2))):
                cur = acc[r, pl.ds(half, 16)]
                acc[r, pl.ds(half, 16)] = jnp.maximum(
                    cur, jnp.where(liota == jj, v, -jnp.inf))

    pltpu.async_copy(chunk_src(0), buf0, sem0).wait()

    def two_chunks(i, _):
        k0 = i * 2
        cp1 = pltpu.async_copy(chunk_src(k0 + 1), buf1, sem1)
        process(buf0)
        cp1.wait()
        nxt = jnp.minimum(k0 + 2, _NCHUNK - 1)
        cp0 = pltpu.async_copy(chunk_src(nxt), buf0, sem0)
        process(buf1)
        cp0.wait()
        return 0

    lax.fori_loop(0, _NCHUNK // 2, two_chunks, 0)
    if _NCHUNK % 2:
        process(buf0)

    pltpu.sync_copy(acc, out_hbm.at[wid])


def _tc_stats_kernel(x_ref, acc_out, g_out, h_out, *, n_steps):
    pi = pl.program_id(0) * pl.num_programs(1) + pl.program_id(1)

    @pl.when(pi == 0)
    def _init():
        acc_out[0:1, :] = jnp.zeros((1, acc_out.shape[1]), jnp.float32)
        acc_out[1:2, :] = jnp.full((1, acc_out.shape[1]), -jnp.inf,
                                   jnp.float32)
        acc_out[2:8, :] = jnp.zeros((6, acc_out.shape[1]), jnp.float32)
        g_out[...] = jnp.zeros_like(g_out)
        h_out[...] = jnp.zeros_like(h_out)

    dn = (((0,), (0,)), ((), ()))
    for i in range(_TC_BB):
        x = x_ref[i]
        a = jnp.exp(-x)
        sig = 1.0 / (1.0 + a)
        om = a * sig
        s2 = sig * sig
        a3 = s2 * sig
        o2 = om * om
        b3 = o2 * om
        a3b = a3.astype(jnp.bfloat16)
        b3b = b3.astype(jnp.bfloat16)
        g_out[...] += jax.lax.dot_general(
            a3b, b3b, dn, preferred_element_type=jnp.float32)
        h_out[...] += jax.lax.dot_general(
            a3b, a3b, dn, preferred_element_type=jnp.float32)
        acc_out[0:1, :] += jnp.sum(a3, axis=0, keepdims=True)
        acc_out[1:2, :] = jnp.maximum(acc_out[1:2, :],
                                      jnp.max(x, axis=0, keepdims=True))


def _combine_kernel(acc_ref, g_ref, h_ref, y_ref, out_ref, *, n_rows):
    nc = acc_ref.shape[1]
    rows = lax.broadcasted_iota(jnp.int32, (nc, nc), 0)
    cols = lax.broadcasted_iota(jnp.int32, (nc, nc), 1)
    q1v = jnp.sum(jnp.where(rows == cols + 10, g_ref[...], 0.0),
                  axis=0, keepdims=True)
    q2v = jnp.sum(jnp.where(rows == cols + 20, g_ref[...], 0.0),
                  axis=0, keepdims=True)
    evc = jnp.sum(jnp.where(rows + 10 == cols, h_ref[...], 0.0),
                  axis=0, keepdims=True)
    ev = jnp.concatenate([evc[:, 20:], evc[:, :20]], axis=1)
    pv = acc_ref[0:1, :]
    mx = acc_ref[1:2, :]
    m1v = jnp.concatenate([mx[:, 10:], mx[:, :10]], axis=1)
    m2v = jnp.concatenate([mx[:, 20:], mx[:, :20]], axis=1)

    y = y_ref[...]
    ssum = jnp.sum(y[:, 0:4, :], axis=0)
    smax = jnp.max(y[:, 4:6, :], axis=0)
    lane80 = lax.broadcasted_iota(jnp.int32, (1, nc), 1)

    def per_class(vals, is_max):
        fill = jnp.full((1, 20), -jnp.inf if is_max else 0.0, jnp.float32)
        return jnp.concatenate(
            [vals[:, 0:10], fill, vals[:, 10:20], fill, fill], axis=1)

    sc_p = per_class(ssum[0:1, :], False)
    sc_q1 = per_class(ssum[1:2, :], False)
    sc_q2 = per_class(ssum[2:3, :], False)
    sc_e = per_class(ssum[3:4, :], False)
    sc_m1 = per_class(smax[0:1, :], True)
    sc_m2 = per_class(smax[1:2, :], True)

    inv_n = 1.0 / n_rows
    p3r = ((pv + sc_p) * inv_n) ** _THIRD
    q1r = ((q1v + sc_q1) * inv_n) ** _THIRD
    q2r = ((q2v + sc_q2) * inv_n) ** _THIRD
    er = ((ev + sc_e) * inv_n) ** _THIRD
    m1 = 1.0 / (1.0 + jnp.exp(-jnp.maximum(m1v, sc_m1)))
    m2 = 1.0 / (1.0 + jnp.exp(-jnp.maximum(m2v, sc_m2)))
    m = jnp.maximum(m1, m2)
    is_src = jnp.logical_or(lane80 < 10,
                            jnp.logical_and(lane80 >= 30, lane80 < 40))
    picked = jnp.where(is_src,
                       0.1 * ((1.0 - m) * p3r + er) + 0.05 * (q1r + q2r),
                       0.0)
    out_ref[...] = jnp.sum(picked, axis=1, keepdims=True)[0:1, 0:1]


def kernel(pred_scores, target_scores):
    del target_scores
    b, a, c = pred_scores.shape
    n_rows = b * a

    mesh = plsc.VectorSubcoreMesh(core_axis_name="c", subcore_axis_name="s")
    sc_stats = pl.kernel(
        _sc_stats_kernel,
        mesh=mesh,
        compiler_params=pltpu.CompilerParams(needs_layout_passes=False),
        out_type=jax.ShapeDtypeStruct((_NW, 8, 32), jnp.float32),
        scratch_types=[
            pltpu.VMEM((_R, c), jnp.float32),
            pltpu.VMEM((_R, c), jnp.float32),
            pltpu.VMEM((8, 32), jnp.float32),
            pltpu.SemaphoreType.DMA,
            pltpu.SemaphoreType.DMA,
        ],
    )(pred_scores)

    gb = b // _TC_BB
    ga = (a - _SC_ROWS) // _TC_AB
    acc, g, h = pl.pallas_call(
        functools.partial(_tc_stats_kernel, n_steps=gb * ga),
        grid=(gb, ga),
        in_specs=[pl.BlockSpec((_TC_BB, _TC_AB, c),
                               lambda i, j: (i, j + 1, 0))],
        out_specs=[
            pl.BlockSpec((8, c), lambda i, j: (0, 0)),
            pl.BlockSpec((c, c), lambda i, j: (0, 0)),
            pl.BlockSpec((c, c), lambda i, j: (0, 0)),
        ],
        out_shape=[
            jax.ShapeDtypeStruct((8, c), jnp.float32),
            jax.ShapeDtypeStruct((c, c), jnp.float32),
            jax.ShapeDtypeStruct((c, c), jnp.float32),
        ],
    )(pred_scores)

    out = pl.pallas_call(
        functools.partial(_combine_kernel, n_rows=n_rows),
        out_shape=jax.ShapeDtypeStruct((1, 1), jnp.float32),
    )(acc, g, h, sc_stats)
    return out.reshape(())
